# baseline (speedup 1.0000x reference)
import jax
import jax.numpy as jnp
from jax.experimental import pallas as pl

_QK_DIM = 96
_TOPK = 4
_TILE = 128
_SCALE = _QK_DIM ** (-0.5)


def _route_kernel(g_ref, wq_ref, bq_ref, wk_ref, bk_ref, w_ref, i_ref):
    g = g_ref[0]
    qh = jax.lax.dot_general(
        g, wq_ref[...], (((1,), (1,)), ((), ())),
        preferred_element_type=jnp.float32) + bq_ref[...]
    kh = jax.lax.dot_general(
        g, wk_ref[...], (((1,), (1,)), ((), ())),
        preferred_element_type=jnp.float32) + bk_ref[...]
    qhs = qh * _SCALE
    n = g.shape[0]
    nt = n // _TILE

    x0s, cms = [], []
    for c in range(nt):
        khc = kh[c * _TILE:(c + 1) * _TILE, :]
        xc = jax.lax.dot_general(
            qhs, khc, (((1,), (1,)), ((), ())),
            preferred_element_type=jnp.float32)
        x0s.append(xc)
        cms.append(jnp.max(xc, axis=1, keepdims=True))

    tile_iota = jax.lax.broadcasted_iota(
        jnp.int32, (_TILE, 1), 0).astype(jnp.float32)
    ones_col = jnp.ones((_TILE, 1), jnp.float32)

    xs = list(x0s)
    cm = jnp.concatenate(cms, axis=1)
    ds, sums, cnts = [], [], []
    for j in range(_TOPK):
        d = jnp.max(cm, axis=1, keepdims=True)
        sc = jnp.zeros((n, 2), jnp.float32)
        new_cms = []
        for c in range(nt):
            hit = xs[c] == d
            hitf = jnp.where(hit, 1.0, 0.0)
            wc = jnp.concatenate(
                [tile_iota + (c * _TILE), ones_col], axis=1)
            sc = sc + jax.lax.dot_general(
                hitf, wc, (((1,), (0,)), ((), ())),
                preferred_element_type=jnp.float32)
            if j + 1 < _TOPK:
                xs[c] = jnp.where(hit, -jnp.inf, xs[c])
                new_cms.append(jnp.max(xs[c], axis=1, keepdims=True))
        ds.append(d)
        sums.append(sc[:, 0:1])
        cnts.append(sc[:, 1:2])
        if j + 1 < _TOPK:
            cm = jnp.concatenate(new_cms, axis=1)
    cnt = jnp.concatenate(cnts, axis=1)
    need_fix = jnp.any(cnt != 1.0)

    @pl.when(jnp.logical_not(need_fix))
    def _fast():
        v = jnp.concatenate(ds, axis=1)
        w = jnp.exp(v - ds[0])
        w_ref[0] = w / jnp.sum(w, axis=1, keepdims=True)
        i_ref[0] = jnp.concatenate(sums, axis=1).astype(jnp.int32)

    @pl.when(need_fix)
    def _exact():
        x0 = jnp.concatenate(x0s, axis=1)
        iota = jax.lax.broadcasted_iota(jnp.int32, x0.shape, 1)
        y = x0
        vals, idxs = [], []
        for j in range(_TOPK):
            m = jnp.max(y, axis=1, keepdims=True)
            idx = jnp.min(jnp.where(y == m, iota, n), axis=1, keepdims=True)
            vals.append(m)
            idxs.append(idx)
            if j + 1 < _TOPK:
                y = jnp.where(iota == idx, -jnp.inf, y)
        v = jnp.concatenate(vals, axis=1)
        w = jnp.exp(v - vals[0])
        w_ref[0] = w / jnp.sum(w, axis=1, keepdims=True)
        i_ref[0] = jnp.concatenate(idxs, axis=1)


@jax.jit
def kernel(g_win, Wq, bq, Wk, bk):
    B, N, D = g_win.shape
    out = pl.pallas_call(
        _route_kernel,
        grid=(B,),
        in_specs=[
            pl.BlockSpec((1, N, D), lambda b: (b, 0, 0)),
            pl.BlockSpec((D, D), lambda b: (0, 0)),
            pl.BlockSpec((1, D), lambda b: (0, 0)),
            pl.BlockSpec((D, D), lambda b: (0, 0)),
            pl.BlockSpec((1, D), lambda b: (0, 0)),
        ],
        out_specs=[
            pl.BlockSpec((1, N, _TOPK), lambda b: (b, 0, 0)),
            pl.BlockSpec((1, N, _TOPK), lambda b: (b, 0, 0)),
        ],
        out_shape=[
            jax.ShapeDtypeStruct((B, N, _TOPK), jnp.float32),
            jax.ShapeDtypeStruct((B, N, _TOPK), jnp.int32),
        ],
    )(g_win, Wq, bq.reshape(1, D), Wk, bk.reshape(1, D))
    return out[0], out[1]


# speedup vs baseline: 1.4643x; 1.4643x over previous
import jax
import jax.numpy as jnp
from jax.experimental import pallas as pl

_QK_DIM = 96
_TOPK = 4
_SCALE = _QK_DIM ** (-0.5)


def _route_kernel(g_ref, wq_ref, bq_ref, wk_ref, bk_ref, w_ref, i_ref):
    g = g_ref[0]
    qh = jax.lax.dot_general(
        g, wq_ref[...], (((1,), (1,)), ((), ())),
        preferred_element_type=jnp.float32) + bq_ref[...]
    kh = jax.lax.dot_general(
        g, wk_ref[...], (((1,), (1,)), ((), ())),
        preferred_element_type=jnp.float32) + bk_ref[...]
    x0 = jax.lax.dot_general(
        qh * _SCALE, kh, (((1,), (1,)), ((), ())),
        preferred_element_type=jnp.float32)
    n = x0.shape[1]

    idx_w = jnp.concatenate(
        [jax.lax.broadcasted_iota(jnp.int32, (n, 1), 0).astype(jnp.float32),
         jnp.ones((n, 1), jnp.float32)], axis=1)
    x = x0
    ds, sums, cnts = [], [], []
    for j in range(_TOPK):
        d = jnp.max(x, axis=1, keepdims=True)
        hit = x == d
        hitf = jnp.where(hit, 1.0, 0.0)
        sc = jax.lax.dot_general(
            hitf, idx_w, (((1,), (0,)), ((), ())),
            preferred_element_type=jnp.float32)
        ds.append(d)
        sums.append(sc[:, 0:1])
        cnts.append(sc[:, 1:2])
        if j + 1 < _TOPK:
            x = jnp.where(hit, -jnp.inf, x)
    cnt = jnp.concatenate(cnts, axis=1)
    need_fix = jnp.any(cnt != 1.0)

    @pl.when(jnp.logical_not(need_fix))
    def _fast():
        v = jnp.concatenate(ds, axis=1)
        w = jnp.exp(v - ds[0])
        w_ref[0] = w / jnp.sum(w, axis=1, keepdims=True)
        i_ref[0] = jnp.concatenate(sums, axis=1).astype(jnp.int32)

    @pl.when(need_fix)
    def _exact():
        iota = jax.lax.broadcasted_iota(jnp.int32, x0.shape, 1)
        y = x0
        vals, idxs = [], []
        for j in range(_TOPK):
            m = jnp.max(y, axis=1, keepdims=True)
            idx = jnp.min(jnp.where(y == m, iota, n), axis=1, keepdims=True)
            vals.append(m)
            idxs.append(idx)
            if j + 1 < _TOPK:
                y = jnp.where(iota == idx, -jnp.inf, y)
        v = jnp.concatenate(vals, axis=1)
        w = jnp.exp(v - vals[0])
        w_ref[0] = w / jnp.sum(w, axis=1, keepdims=True)
        i_ref[0] = jnp.concatenate(idxs, axis=1)


@jax.jit
def kernel(g_win, Wq, bq, Wk, bk):
    B, N, D = g_win.shape
    out = pl.pallas_call(
        _route_kernel,
        grid=(B,),
        in_specs=[
            pl.BlockSpec((1, N, D), lambda b: (b, 0, 0)),
            pl.BlockSpec((D, D), lambda b: (0, 0)),
            pl.BlockSpec((1, D), lambda b: (0, 0)),
            pl.BlockSpec((D, D), lambda b: (0, 0)),
            pl.BlockSpec((1, D), lambda b: (0, 0)),
        ],
        out_specs=[
            pl.BlockSpec((1, N, _TOPK), lambda b: (b, 0, 0)),
            pl.BlockSpec((1, N, _TOPK), lambda b: (b, 0, 0)),
        ],
        out_shape=[
            jax.ShapeDtypeStruct((B, N, _TOPK), jnp.float32),
            jax.ShapeDtypeStruct((B, N, _TOPK), jnp.int32),
        ],
    )(g_win, Wq, bq.reshape(1, D), Wk, bk.reshape(1, D))
    return out[0], out[1]


# speedup vs baseline: 1.5198x; 1.0379x over previous
import jax
import jax.numpy as jnp
from jax.experimental import pallas as pl

_QK_DIM = 96
_TOPK = 4
_SCALE = _QK_DIM ** (-0.5)


def _route_kernel(g_ref, wq_ref, bq_ref, wk_ref, bk_ref, w_ref, i_ref):
    g = g_ref[0]
    qh = jax.lax.dot_general(
        g, wq_ref[...], (((1,), (1,)), ((), ())),
        preferred_element_type=jnp.float32) + bq_ref[...]
    kh = jax.lax.dot_general(
        g, wk_ref[...], (((1,), (1,)), ((), ())),
        preferred_element_type=jnp.float32) + bk_ref[...]
    x0 = jax.lax.dot_general(
        qh * _SCALE, kh, (((1,), (1,)), ((), ())),
        preferred_element_type=jnp.float32)
    n = x0.shape[1]

    idx_w = jnp.concatenate(
        [jax.lax.broadcasted_iota(jnp.int32, (n, 1), 0).astype(jnp.float32),
         jnp.ones((n, 1), jnp.float32)], axis=1)
    x = x0
    ds, sums, cnts = [], [], []
    hitf_prev = None
    for j in range(_TOPK):
        d = jnp.max(x, axis=1, keepdims=True)
        hit = x == d
        hitf = jnp.where(hit, 1.0 if j % 2 == 0 else 2048.0, 0.0)
        ds.append(d)
        if j % 2 == 0:
            hitf_prev = hitf
        else:
            sc = jax.lax.dot_general(
                hitf_prev + hitf, idx_w, (((1,), (0,)), ((), ())),
                preferred_element_type=jnp.float32)
            hi = jnp.floor(sc * (1.0 / 2048.0))
            lo = sc - 2048.0 * hi
            sums.extend([lo[:, 0:1], hi[:, 0:1]])
            cnts.extend([lo[:, 1:2], hi[:, 1:2]])
        if j + 1 < _TOPK:
            x = jnp.where(hit, -jnp.inf, x)
    cnt = jnp.concatenate(cnts, axis=1)
    need_fix = jnp.any(cnt != 1.0)

    @pl.when(jnp.logical_not(need_fix))
    def _fast():
        v = jnp.concatenate(ds, axis=1)
        w = jnp.exp(v - ds[0])
        w_ref[0] = w / jnp.sum(w, axis=1, keepdims=True)
        i_ref[0] = jnp.concatenate(sums, axis=1).astype(jnp.int32)

    @pl.when(need_fix)
    def _exact():
        iota = jax.lax.broadcasted_iota(jnp.int32, x0.shape, 1)
        y = x0
        vals, idxs = [], []
        for j in range(_TOPK):
            m = jnp.max(y, axis=1, keepdims=True)
            idx = jnp.min(jnp.where(y == m, iota, n), axis=1, keepdims=True)
            vals.append(m)
            idxs.append(idx)
            if j + 1 < _TOPK:
                y = jnp.where(iota == idx, -jnp.inf, y)
        v = jnp.concatenate(vals, axis=1)
        w = jnp.exp(v - vals[0])
        w_ref[0] = w / jnp.sum(w, axis=1, keepdims=True)
        i_ref[0] = jnp.concatenate(idxs, axis=1)


@jax.jit
def kernel(g_win, Wq, bq, Wk, bk):
    B, N, D = g_win.shape
    out = pl.pallas_call(
        _route_kernel,
        grid=(B,),
        in_specs=[
            pl.BlockSpec((1, N, D), lambda b: (b, 0, 0)),
            pl.BlockSpec((D, D), lambda b: (0, 0)),
            pl.BlockSpec((1, D), lambda b: (0, 0)),
            pl.BlockSpec((D, D), lambda b: (0, 0)),
            pl.BlockSpec((1, D), lambda b: (0, 0)),
        ],
        out_specs=[
            pl.BlockSpec((1, N, _TOPK), lambda b: (b, 0, 0)),
            pl.BlockSpec((1, N, _TOPK), lambda b: (b, 0, 0)),
        ],
        out_shape=[
            jax.ShapeDtypeStruct((B, N, _TOPK), jnp.float32),
            jax.ShapeDtypeStruct((B, N, _TOPK), jnp.int32),
        ],
    )(g_win, Wq, bq.reshape(1, D), Wk, bk.reshape(1, D))
    return out[0], out[1]


# speedup vs baseline: 1.5700x; 1.0330x over previous
import jax
import jax.numpy as jnp
from jax.experimental import pallas as pl

_QK_DIM = 96
_TOPK = 4
_SCALE = _QK_DIM ** (-0.5)


def _route_kernel(g_ref, wq_ref, bq_ref, wk_ref, bk_ref, w_ref, i_ref):
    g = g_ref[0]
    qh = jax.lax.dot_general(
        g, wq_ref[...], (((1,), (1,)), ((), ())),
        preferred_element_type=jnp.float32) + bq_ref[...]
    kh = jax.lax.dot_general(
        g, wk_ref[...], (((1,), (1,)), ((), ())),
        preferred_element_type=jnp.float32) + bk_ref[...]
    x0 = jax.lax.dot_general(
        qh * _SCALE, kh, (((1,), (1,)), ((), ())),
        preferred_element_type=jnp.float32)
    n = x0.shape[1]

    idx_w = jnp.concatenate(
        [jax.lax.broadcasted_iota(jnp.int32, (n, 1), 0).astype(jnp.float32),
         jnp.ones((n, 1), jnp.float32)], axis=1)
    x = x0
    ds, sums, cnts = [], [], []
    hitf_prev = None
    for j in range(_TOPK):
        d = jnp.max(x, axis=1, keepdims=True)
        hit = x == d
        ds.append(d)
        if j % 2 == 0:
            hitf_prev = jnp.where(hit, 1.0, 0.0)
        else:
            packed = jnp.where(hit, 2048.0, hitf_prev)
            sc = jax.lax.dot_general(
                packed, idx_w, (((1,), (0,)), ((), ())),
                preferred_element_type=jnp.float32)
            hi = jnp.floor(sc * (1.0 / 2048.0))
            lo = sc - 2048.0 * hi
            sums.extend([lo[:, 0:1], hi[:, 0:1]])
            cnts.extend([lo[:, 1:2], hi[:, 1:2]])
        if j + 1 < _TOPK:
            x = jnp.where(hit, -jnp.inf, x)
    cnt = jnp.concatenate(cnts, axis=1)
    need_fix = jnp.any(cnt != 1.0)

    @pl.when(jnp.logical_not(need_fix))
    def _fast():
        v = jnp.concatenate(ds, axis=1)
        w = jnp.exp(v - ds[0])
        w_ref[0] = w / jnp.sum(w, axis=1, keepdims=True)
        i_ref[0] = jnp.concatenate(sums, axis=1).astype(jnp.int32)

    @pl.when(need_fix)
    def _exact():
        iota = jax.lax.broadcasted_iota(jnp.int32, x0.shape, 1)
        y = x0
        vals, idxs = [], []
        for j in range(_TOPK):
            m = jnp.max(y, axis=1, keepdims=True)
            idx = jnp.min(jnp.where(y == m, iota, n), axis=1, keepdims=True)
            vals.append(m)
            idxs.append(idx)
            if j + 1 < _TOPK:
                y = jnp.where(iota == idx, -jnp.inf, y)
        v = jnp.concatenate(vals, axis=1)
        w = jnp.exp(v - vals[0])
        w_ref[0] = w / jnp.sum(w, axis=1, keepdims=True)
        i_ref[0] = jnp.concatenate(idxs, axis=1)


@jax.jit
def kernel(g_win, Wq, bq, Wk, bk):
    B, N, D = g_win.shape
    out = pl.pallas_call(
        _route_kernel,
        grid=(B,),
        in_specs=[
            pl.BlockSpec((1, N, D), lambda b: (b, 0, 0)),
            pl.BlockSpec((D, D), lambda b: (0, 0)),
            pl.BlockSpec((1, D), lambda b: (0, 0)),
            pl.BlockSpec((D, D), lambda b: (0, 0)),
            pl.BlockSpec((1, D), lambda b: (0, 0)),
        ],
        out_specs=[
            pl.BlockSpec((1, N, _TOPK), lambda b: (b, 0, 0)),
            pl.BlockSpec((1, N, _TOPK), lambda b: (b, 0, 0)),
        ],
        out_shape=[
            jax.ShapeDtypeStruct((B, N, _TOPK), jnp.float32),
            jax.ShapeDtypeStruct((B, N, _TOPK), jnp.int32),
        ],
    )(g_win, Wq, bq.reshape(1, D), Wk, bk.reshape(1, D))
    return out[0], out[1]
